# baseline (speedup 1.0000x reference)
import jax
import jax.numpy as jnp
from jax import lax
from jax.experimental import pallas as pl
from jax.experimental.pallas import tpu as pltpu

NBINS = 256
ROWS = 128
N = 64 * 64 * 64
CENTER = N // 2
CHUNK = 4096
NSTEPS = N // CHUNK
CENTER_STEP = CENTER // CHUNK
NORM = 65 * 65 * 65
LOG2E = 1.4426950408889634


def _entropy_body(x_ref, out_ref, acc_ref, cen_ref):
    i = pl.program_id(0)

    @pl.when(i == 0)
    def _():
        acc_ref[...] = jnp.zeros_like(acc_ref)

    blk = x_ref[...]
    acc_ref[...] += blk.reshape(ROWS, CHUNK // 128, 128).sum(axis=1)

    @pl.when(i == CENTER_STEP)
    def _():
        cen_ref[...] = blk[:, 0:1]

    @pl.when(i == NSTEPS - 1)
    def _():
        sums = acc_ref[...].sum(axis=1, keepdims=True)
        cen = cen_ref[...]
        nb = (sums - cen) * (1.0 / (N - 1))
        vals = jnp.concatenate([cen, nb], axis=0)
        idx = jnp.floor(vals * NBINS).astype(jnp.int32)
        valid = (vals >= 0.0) & (vals <= 1.0)
        idx = jnp.minimum(idx, NBINS - 1)
        bins = lax.broadcasted_iota(jnp.int32, (2 * ROWS, NBINS), 1)
        match = (idx == bins) & valid
        counts = jnp.sum(match.astype(jnp.float32), axis=0, keepdims=True)
        p = counts * (1.0 / NORM)
        e = -jnp.sum(p * (jnp.log(p + 1e-10) * LOG2E), axis=1, keepdims=True)
        out_ref[...] = e


def kernel(F):
    x = F.reshape(ROWS, N)
    out = pl.pallas_call(
        _entropy_body,
        grid=(NSTEPS,),
        in_specs=[pl.BlockSpec((ROWS, CHUNK), lambda i: (0, i))],
        out_specs=pl.BlockSpec((1, 1), lambda i: (0, 0)),
        out_shape=jax.ShapeDtypeStruct((1, 1), jnp.float32),
        scratch_shapes=[
            pltpu.VMEM((ROWS, 128), jnp.float32),
            pltpu.VMEM((ROWS, 1), jnp.float32),
        ],
    )(x)
    return out.reshape(())


# speedup vs baseline: 1.0669x; 1.0669x over previous
import jax
import jax.numpy as jnp
from jax import lax
from jax.experimental import pallas as pl
from jax.experimental.pallas import tpu as pltpu

NBINS = 256
ROWS = 128
N = 64 * 64 * 64
SUBL = N // 128
CENTER_SUBL = (N // 2) // 128
RPB = 2
NSTEPS = ROWS // RPB
NORM = 65 * 65 * 65
LOG2E = 1.4426950408889634


def _entropy_body(x_ref, out_ref, acc_ref, cen_ref):
    i = pl.program_id(0)

    blk = x_ref[...]
    acc_ref[pl.ds(i * RPB, RPB), :] = blk.sum(axis=1)
    cen_ref[pl.ds(i * RPB, RPB), :] = blk[:, CENTER_SUBL, 0:1]

    @pl.when(i == NSTEPS - 1)
    def _():
        sums = acc_ref[...].sum(axis=1, keepdims=True)
        cen = cen_ref[...]
        nb = (sums - cen) * (1.0 / (N - 1))
        vals = jnp.concatenate([cen, nb], axis=0)
        idx = jnp.floor(vals * NBINS).astype(jnp.int32)
        valid = (vals >= 0.0) & (vals <= 1.0)
        idx = jnp.minimum(idx, NBINS - 1)
        bins = lax.broadcasted_iota(jnp.int32, (2 * ROWS, NBINS), 1)
        match = (idx == bins) & valid
        counts = jnp.sum(match.astype(jnp.float32), axis=0, keepdims=True)
        p = counts * (1.0 / NORM)
        e = -jnp.sum(p * (jnp.log(p + 1e-10) * LOG2E), axis=1, keepdims=True)
        out_ref[...] = e


def kernel(F):
    x = F.reshape(ROWS, SUBL, 128)
    out = pl.pallas_call(
        _entropy_body,
        grid=(NSTEPS,),
        in_specs=[pl.BlockSpec((RPB, SUBL, 128), lambda i: (i, 0, 0))],
        out_specs=pl.BlockSpec((1, 1), lambda i: (0, 0)),
        out_shape=jax.ShapeDtypeStruct((1, 1), jnp.float32),
        scratch_shapes=[
            pltpu.VMEM((ROWS, 128), jnp.float32),
            pltpu.VMEM((ROWS, 1), jnp.float32),
        ],
    )(x)
    return out.reshape(())
